# baseline (speedup 1.0000x reference)
import jax
import jax.numpy as jnp
from jax.experimental import pallas as pl

_BBLK = 1024


def _matmul_kernel(cont_ref, m_ref, out_ref):
    out_ref[...] = jax.lax.dot_general(
        cont_ref[...], m_ref[...],
        dimension_numbers=(((1,), (0,)), ((), ())),
        preferred_element_type=jnp.float32,
        precision=jax.lax.Precision.DEFAULT,
    )


def kernel(cont, weight):
    B, F = cont.shape
    _, E = weight.shape
    f_idx = jnp.arange(F)[:, None]
    col_f = jnp.arange(F * E)[None, :] // E
    m = (f_idx == col_f).astype(weight.dtype) * weight.reshape(1, F * E)

    out2d = pl.pallas_call(
        _matmul_kernel,
        grid=(B // _BBLK,),
        in_specs=[
            pl.BlockSpec((_BBLK, F), lambda i: (i, 0)),
            pl.BlockSpec((F, F * E), lambda i: (0, 0)),
        ],
        out_specs=pl.BlockSpec((_BBLK, F * E), lambda i: (i, 0)),
        out_shape=jax.ShapeDtypeStruct((B, F * E), cont.dtype),
    )(cont, m)
    return out2d.reshape(B, F, E)
